# Initial kernel scaffold; baseline (speedup 1.0000x reference)
import jax
import jax.numpy as jnp
from jax.experimental import pallas as pl


def kernel(h, e, u, bond_src, bond_dst, atom_graph, bond_graph, snorm_n, snorm_e, WA, bA, WB, bB, WC, bC, WD, bD, WE, bE, WF, bF, WG, bG, WH, bH, WI, bI, gamma_h, beta_h, gamma_e, beta_e, gamma_u, beta_u):
    raise NotImplementedError("write your pallas kernel here")


# speedup vs baseline: 3.7800x; 3.7800x over previous
import functools

import jax
import jax.numpy as jnp
from jax import lax
from jax.experimental import pallas as pl
from jax.experimental.pallas import tpu as pltpu
from jax.experimental.pallas import tpu_sc as plsc

NA = 10000
NB = 320000
NG = 64
D = 128
H = 64

NC = 2
NS = 16
NW = NC * NS
EPW = NB // NW
K = 40
CHUNKS = EPW // K
RPS = NA // NS


def _k0_body(u_ref, w_ref, b_ref, out_ref):
    out_ref[...] = jnp.dot(u_ref[...], w_ref[...],
                           preferred_element_type=jnp.float32) + b_ref[...]


def _u_proj(u, Wcat, bcat):
    return pl.pallas_call(
        _k0_body,
        out_shape=jax.ShapeDtypeStruct((NG, Wcat.shape[1]), jnp.float32),
    )(u, Wcat, bcat)


def _k1_body(h_ref, ag_ref, w_ref, b_ref, dh_ref, aeh0_ref, aeh1_ref,
             segg_ref, cnta_ref):
    y = jnp.dot(h_ref[...], w_ref[...],
                preferred_element_type=jnp.float32) + b_ref[...]
    dh_ref[...] = y[:, D:2 * D]
    aeh0_ref[...] = jnp.concatenate(
        [y[:, 0:H], y[:, 2 * D:2 * D + H]], axis=1)
    aeh1_ref[...] = jnp.concatenate(
        [y[:, H:D], y[:, 2 * D + H:3 * D]], axis=1)
    ag = ag_ref[...]
    onehot = (ag == lax.broadcasted_iota(jnp.int32, (1, NG), 1)
              ).astype(jnp.float32)
    ones = jnp.ones((h_ref.shape[0], D), jnp.float32)
    dn = (((0,), (0,)), ((), ()))
    seg = lax.dot_general(onehot, y[:, 3 * D:4 * D], dn,
                          preferred_element_type=jnp.float32)
    cnt = lax.dot_general(onehot, ones, dn,
                          preferred_element_type=jnp.float32)

    @pl.when(pl.program_id(0) == 0)
    def _():
        segg_ref[...] = jnp.zeros_like(segg_ref)
        cnta_ref[...] = jnp.zeros_like(cnta_ref)

    segg_ref[...] += seg
    cnta_ref[...] += cnt


def _h_proj(h, ag2d, Wcat, bcat):
    R = 1000
    grid = NA // R
    return pl.pallas_call(
        _k1_body,
        grid=(grid,),
        in_specs=[
            pl.BlockSpec((R, D), lambda i: (i, 0)),
            pl.BlockSpec((R, 1), lambda i: (i, 0)),
            pl.BlockSpec((D, 4 * D), lambda i: (0, 0)),
            pl.BlockSpec((1, 4 * D), lambda i: (0, 0)),
        ],
        out_specs=[
            pl.BlockSpec((R, D), lambda i: (i, 0)),
            pl.BlockSpec((R, D), lambda i: (i, 0)),
            pl.BlockSpec((R, D), lambda i: (i, 0)),
            pl.BlockSpec((NG, D), lambda i: (0, 0)),
            pl.BlockSpec((NG, D), lambda i: (0, 0)),
        ],
        out_shape=[
            jax.ShapeDtypeStruct((NA, D), jnp.float32),
            jax.ShapeDtypeStruct((NA, D), jnp.float32),
            jax.ShapeDtypeStruct((NA, D), jnp.float32),
            jax.ShapeDtypeStruct((NG, D), jnp.float32),
            jax.ShapeDtypeStruct((NG, D), jnp.float32),
        ],
    )(h, ag2d, Wcat, bcat)


def _k2_body(e_ref, bg_ref, w_ref, b_ref, cu_ref,
             becu0_ref, becu1_ref, segh_ref, cntb_ref):
    y = jnp.dot(e_ref[...], w_ref[...],
                preferred_element_type=jnp.float32) + b_ref[...]
    bg = bg_ref[...]
    onehot = (bg == lax.broadcasted_iota(jnp.int32, (1, NG), 1)
              ).astype(jnp.float32)
    becu = y[:, :D] + jnp.dot(onehot, cu_ref[...],
                              preferred_element_type=jnp.float32)
    becu0_ref[...] = becu[:, :H]
    becu1_ref[...] = becu[:, H:]
    he = y[:, D:]
    ones = jnp.ones((e_ref.shape[0], D), jnp.float32)
    dn = (((0,), (0,)), ((), ()))
    seg = lax.dot_general(onehot, he, dn, preferred_element_type=jnp.float32)
    cnt = lax.dot_general(onehot, ones, dn, preferred_element_type=jnp.float32)

    @pl.when(pl.program_id(0) == 0)
    def _():
        segh_ref[...] = jnp.zeros_like(segh_ref)
        cntb_ref[...] = jnp.zeros_like(cntb_ref)

    segh_ref[...] += seg
    cntb_ref[...] += cnt


def _e_proj(e, bg2d, Wcat, bcat, Cu):
    R = 4000
    grid = NB // R
    return pl.pallas_call(
        _k2_body,
        grid=(grid,),
        in_specs=[
            pl.BlockSpec((R, D), lambda i: (i, 0)),
            pl.BlockSpec((R, 1), lambda i: (i, 0)),
            pl.BlockSpec((D, 2 * D), lambda i: (0, 0)),
            pl.BlockSpec((1, 2 * D), lambda i: (0, 0)),
            pl.BlockSpec((NG, D), lambda i: (0, 0)),
        ],
        out_specs=[
            pl.BlockSpec((R, H), lambda i: (i, 0)),
            pl.BlockSpec((R, H), lambda i: (i, 0)),
            pl.BlockSpec((NG, D), lambda i: (0, 0)),
            pl.BlockSpec((NG, D), lambda i: (0, 0)),
        ],
        out_shape=[
            jax.ShapeDtypeStruct((NB, H), jnp.float32),
            jax.ShapeDtypeStruct((NB, H), jnp.float32),
            jax.ShapeDtypeStruct((NG, D), jnp.float32),
            jax.ShapeDtypeStruct((NG, D), jnp.float32),
        ],
    )(e, bg2d, Wcat, bcat, Cu)


def _sc_body(aeh0, aeh1, becu0, becu1, src_hbm, dst_hbm, zeros_hbm,
             ep0, ep1, nd0, nd1,
             idx_s, idx_d, aes, aed, becu_v, ep_v, bufs, bufd,
             accum, sem0, sem1, sem2):
    cid = lax.axis_index("c")
    sid = lax.axis_index("s")
    wid = sid * NC + cid
    edge_base = wid * EPW

    for half, (aeh, becu, ep_out, nd_out) in enumerate(
            ((aeh0, becu0, ep0, nd0), (aeh1, becu1, ep1, nd1))):
        @pl.when(sid == 0)
        def _():
            pltpu.sync_copy(zeros_hbm, accum)
        plsc.subcore_barrier()

        def chunk(c, carry):
            base = edge_base + c * K
            pltpu.sync_copy(src_hbm.at[pl.ds(base, K)], idx_s)
            pltpu.sync_copy(dst_hbm.at[pl.ds(base, K)], idx_d)
            c0 = pltpu.async_copy(aeh.at[idx_s], aes, sem0)
            c1 = pltpu.async_copy(aeh.at[idx_d], aed, sem1)
            c2 = pltpu.async_copy(becu.at[pl.ds(base, K)], becu_v, sem2)
            c0.wait(); c1.wait(); c2.wait()

            def row(i, carry2):
                def col(j, carry3):
                    s = pl.ds(j * 16, 16)
                    s2 = pl.ds(H + j * 16, 16)
                    a = aes[i, s] + aed[i, s] + becu_v[i, s]
                    ep_v[i, s] = a
                    sg = 1.0 / (1.0 + jnp.exp(-a))
                    bufs[i, s] = sg * aed[i, s2]
                    bufd[i, s] = sg * aes[i, s2]
                    bufs[i, s2] = sg
                    bufd[i, s2] = sg
                    return carry3
                return lax.fori_loop(0, H // 16, col, carry2)
            lax.fori_loop(0, K, row, 0)

            pltpu.sync_copy(ep_v, ep_out.at[pl.ds(base, K)])
            pltpu.sync_copy(bufs, accum.at[idx_s], add=True)
            pltpu.sync_copy(bufd, accum.at[idx_d], add=True)
            return carry

        lax.fori_loop(0, CHUNKS, chunk, 0)
        plsc.subcore_barrier()

        @pl.when(sid == 0)
        def _():
            pltpu.sync_copy(accum, nd_out.at[pl.ds(cid * NA, NA)])
        plsc.subcore_barrier()


def _sc_edges(aeh0, aeh1, becu0, becu1, src, dst, zeros):
    fn = pl.kernel(
        _sc_body,
        out_type=[
            jax.ShapeDtypeStruct((NB, H), jnp.float32),
            jax.ShapeDtypeStruct((NB, H), jnp.float32),
            jax.ShapeDtypeStruct((NC * NA, D), jnp.float32),
            jax.ShapeDtypeStruct((NC * NA, D), jnp.float32),
        ],
        mesh=plsc.VectorSubcoreMesh(core_axis_name="c", subcore_axis_name="s"),
        scratch_types=[
            pltpu.VMEM((K,), jnp.int32),
            pltpu.VMEM((K,), jnp.int32),
            pltpu.VMEM((K, D), jnp.float32),
            pltpu.VMEM((K, D), jnp.float32),
            pltpu.VMEM((K, H), jnp.float32),
            pltpu.VMEM((K, H), jnp.float32),
            pltpu.VMEM((K, D), jnp.float32),
            pltpu.VMEM((K, D), jnp.float32),
            pltpu.VMEM_SHARED((NA, D), jnp.float32),
            pltpu.SemaphoreType.DMA,
            pltpu.SemaphoreType.DMA,
            pltpu.SemaphoreType.DMA,
        ],
    )
    return fn(aeh0, aeh1, becu0, becu1, src, dst, zeros)


def _k4_body(nd0a_ref, nd0b_ref, nd1a_ref, nd1b_ref, dh_ref, ag_ref,
             fu_ref, sn_ref, hpre_ref, stats_ref):
    num = jnp.concatenate(
        [nd0a_ref[:, :H] + nd0b_ref[:, :H],
         nd1a_ref[:, :H] + nd1b_ref[:, :H]], axis=1)
    den = jnp.concatenate(
        [nd0a_ref[:, H:] + nd0b_ref[:, H:],
         nd1a_ref[:, H:] + nd1b_ref[:, H:]], axis=1)
    gated = num / (den + 1e-6)
    ag = ag_ref[...]
    onehot = (ag == lax.broadcasted_iota(jnp.int32, (1, NG), 1)
              ).astype(jnp.float32)
    fug = jnp.dot(onehot, fu_ref[...], preferred_element_type=jnp.float32)
    x = (dh_ref[...] + gated + fug) * sn_ref[...]
    hpre_ref[...] = x
    s1 = jnp.sum(x, axis=0, keepdims=True)
    s2 = jnp.sum(x * x, axis=0, keepdims=True)
    st = jnp.concatenate([s1, s2, jnp.zeros((6, D), jnp.float32)], axis=0)

    @pl.when(pl.program_id(0) == 0)
    def _():
        stats_ref[...] = jnp.zeros_like(stats_ref)

    stats_ref[...] += st


def _h_assemble(nd0, nd1, Dh, ag2d, Fu, snorm_n):
    R = 1000
    grid = NA // R
    return pl.pallas_call(
        _k4_body,
        grid=(grid,),
        in_specs=[
            pl.BlockSpec((R, D), lambda i: (i, 0)),
            pl.BlockSpec((R, D), lambda i: (i + NA // R, 0)),
            pl.BlockSpec((R, D), lambda i: (i, 0)),
            pl.BlockSpec((R, D), lambda i: (i + NA // R, 0)),
            pl.BlockSpec((R, D), lambda i: (i, 0)),
            pl.BlockSpec((R, 1), lambda i: (i, 0)),
            pl.BlockSpec((NG, D), lambda i: (0, 0)),
            pl.BlockSpec((R, 1), lambda i: (i, 0)),
        ],
        out_specs=[
            pl.BlockSpec((R, D), lambda i: (i, 0)),
            pl.BlockSpec((8, D), lambda i: (0, 0)),
        ],
        out_shape=[
            jax.ShapeDtypeStruct((NA, D), jnp.float32),
            jax.ShapeDtypeStruct((8, D), jnp.float32),
        ],
    )(nd0, nd0, nd1, nd1, Dh, ag2d, Fu, snorm_n)


def _k5_body(ep0_ref, ep1_ref, sn_ref, stats_ref):
    x = jnp.concatenate([ep0_ref[...], ep1_ref[...]], axis=1) * sn_ref[...]
    s1 = jnp.sum(x, axis=0, keepdims=True)
    s2 = jnp.sum(x * x, axis=0, keepdims=True)
    st = jnp.concatenate([s1, s2, jnp.zeros((6, D), jnp.float32)], axis=0)

    @pl.when(pl.program_id(0) == 0)
    def _():
        stats_ref[...] = jnp.zeros_like(stats_ref)

    stats_ref[...] += st


def _e_stats(ep0, ep1, snorm_e):
    R = 4000
    grid = NB // R
    return pl.pallas_call(
        _k5_body,
        grid=(grid,),
        in_specs=[
            pl.BlockSpec((R, H), lambda i: (i, 0)),
            pl.BlockSpec((R, H), lambda i: (i, 0)),
            pl.BlockSpec((R, 1), lambda i: (i, 0)),
        ],
        out_specs=pl.BlockSpec((8, D), lambda i: (0, 0)),
        out_shape=jax.ShapeDtypeStruct((8, D), jnp.float32),
    )(ep0, ep1, snorm_e)


def _bn_elu(x, stats, n, gamma, beta):
    m = stats[0:1, :] / n
    v = stats[1:2, :] / n - m * m
    y = gamma * (x - m) * lax.rsqrt(v + 1e-5) + beta
    return jnp.where(y > 0, y, jnp.exp(jnp.minimum(y, 0.0)) - 1.0)


def _k6_body(ep0_ref, ep1_ref, sn_ref, stats_ref, g_ref, b_ref, out_ref):
    x = jnp.concatenate([ep0_ref[...], ep1_ref[...]], axis=1) * sn_ref[...]
    out_ref[...] = _bn_elu(x, stats_ref[...], float(NB), g_ref[...], b_ref[...])


def _e_apply(ep0, ep1, snorm_e, stats, gamma, beta):
    R = 4000
    grid = NB // R
    return pl.pallas_call(
        _k6_body,
        grid=(grid,),
        in_specs=[
            pl.BlockSpec((R, H), lambda i: (i, 0)),
            pl.BlockSpec((R, H), lambda i: (i, 0)),
            pl.BlockSpec((R, 1), lambda i: (i, 0)),
            pl.BlockSpec((8, D), lambda i: (0, 0)),
            pl.BlockSpec((1, D), lambda i: (0, 0)),
            pl.BlockSpec((1, D), lambda i: (0, 0)),
        ],
        out_specs=pl.BlockSpec((R, D), lambda i: (i, 0)),
        out_shape=jax.ShapeDtypeStruct((NB, D), jnp.float32),
    )(ep0, ep1, snorm_e, stats, gamma, beta)


def _k7a_body(x_ref, stats_ref, g_ref, b_ref, out_ref):
    out_ref[...] = _bn_elu(x_ref[...], stats_ref[...], float(NA),
                           g_ref[...], b_ref[...])


def _h_apply(hpre, stats, gamma, beta):
    R = 1000
    grid = NA // R
    return pl.pallas_call(
        _k7a_body,
        grid=(grid,),
        in_specs=[
            pl.BlockSpec((R, D), lambda i: (i, 0)),
            pl.BlockSpec((8, D), lambda i: (0, 0)),
            pl.BlockSpec((1, D), lambda i: (0, 0)),
            pl.BlockSpec((1, D), lambda i: (0, 0)),
        ],
        out_specs=pl.BlockSpec((R, D), lambda i: (i, 0)),
        out_shape=jax.ShapeDtypeStruct((NA, D), jnp.float32),
    )(hpre, stats, gamma, beta)


def _k7b_body(u_ref, wi_ref, bi_ref, segg_ref, cnta_ref, segh_ref, cntb_ref,
              g_ref, b_ref, out_ref):
    iu = jnp.dot(u_ref[...], wi_ref[...],
                 preferred_element_type=jnp.float32) + bi_ref[...]
    x = (segg_ref[...] / jnp.maximum(cnta_ref[...], 1.0)
         + segh_ref[...] / jnp.maximum(cntb_ref[...], 1.0) + iu)
    m = jnp.mean(x, axis=0, keepdims=True)
    v = jnp.mean(x * x, axis=0, keepdims=True) - m * m
    y = g_ref[...] * (x - m) * lax.rsqrt(v + 1e-5) + b_ref[...]
    out_ref[...] = jnp.where(y > 0, y, jnp.exp(jnp.minimum(y, 0.0)) - 1.0)


def _u_update(u, WI, bI, segG, cntA, segH, cntB, gamma, beta):
    return pl.pallas_call(
        _k7b_body,
        out_shape=jax.ShapeDtypeStruct((NG, D), jnp.float32),
    )(u, WI, bI, segG, cntA, segH, cntB, gamma, beta)


@jax.jit
def kernel(h, e, u, bond_src, bond_dst, atom_graph, bond_graph, snorm_n,
           snorm_e, WA, bA, WB, bB, WC, bC, WD, bD, WE, bE, WF, bF, WG, bG,
           WH, bH, WI, bI, gamma_h, beta_h, gamma_e, beta_e, gamma_u, beta_u):
    ag2d = atom_graph.reshape(NA, 1)
    bg2d = bond_graph.reshape(NB, 1)

    cufu = _u_proj(u, jnp.concatenate([WC, WF], axis=1),
                   jnp.concatenate([bC, bF]).reshape(1, 2 * D))
    Cu = cufu[:, :D]
    Fu = cufu[:, D:]

    Wh = jnp.concatenate([WA, WD, WE, WG], axis=1)
    bh = jnp.concatenate([bA, bD, bE, bG]).reshape(1, 4 * D)
    Dh, aeh0, aeh1, segG, cntA = _h_proj(h, ag2d, Wh, bh)

    We = jnp.concatenate([WB, WH], axis=1)
    be = jnp.concatenate([bB, bH]).reshape(1, 2 * D)
    becu0, becu1, segH, cntB = _e_proj(e, bg2d, We, be, Cu)

    zeros = jnp.zeros((NA, D), jnp.float32)
    ep0, ep1, nd0, nd1 = _sc_edges(aeh0, aeh1, becu0, becu1,
                                   bond_src, bond_dst, zeros)

    hpre, hstats = _h_assemble(nd0, nd1, Dh, ag2d, Fu, snorm_n)

    estats = _e_stats(ep0, ep1, snorm_e)
    e_new = _e_apply(ep0, ep1, snorm_e, estats,
                     gamma_e.reshape(1, D), beta_e.reshape(1, D))

    h_new = _h_apply(hpre, hstats, gamma_h.reshape(1, D), beta_h.reshape(1, D))

    u_new = _u_update(u, WI, bI.reshape(1, D), segG, cntA, segH, cntB,
                      gamma_u.reshape(1, D), beta_u.reshape(1, D))

    return h_new, e_new, u_new


# speedup vs baseline: 4.8690x; 1.2881x over previous
import functools

import jax
import jax.numpy as jnp
from jax import lax
from jax.experimental import pallas as pl
from jax.experimental.pallas import tpu as pltpu
from jax.experimental.pallas import tpu_sc as plsc

NA = 10000
NB = 320000
NG = 64
D = 128
H = 64

NC = 2
NS = 16
NW = NC * NS
EPW = NB // NW
K = 40
CHUNKS = EPW // K
RPS = NA // NS


def _k0_body(u_ref, w_ref, b_ref, out_ref):
    out_ref[...] = jnp.dot(u_ref[...], w_ref[...],
                           preferred_element_type=jnp.float32) + b_ref[...]


def _u_proj(u, Wcat, bcat):
    return pl.pallas_call(
        _k0_body,
        out_shape=jax.ShapeDtypeStruct((NG, Wcat.shape[1]), jnp.float32),
    )(u, Wcat, bcat)


def _k1_body(h_ref, ag_ref, w_ref, b_ref, dh_ref, aeh0_ref, aeh1_ref,
             segg_ref, cnta_ref):
    y = jnp.dot(h_ref[...], w_ref[...],
                preferred_element_type=jnp.float32) + b_ref[...]
    dh_ref[...] = y[:, D:2 * D]
    aeh0_ref[...] = jnp.concatenate(
        [y[:, 0:H], y[:, 2 * D:2 * D + H]], axis=1)
    aeh1_ref[...] = jnp.concatenate(
        [y[:, H:D], y[:, 2 * D + H:3 * D]], axis=1)
    ag = ag_ref[...]
    onehot = (ag == lax.broadcasted_iota(jnp.int32, (1, NG), 1)
              ).astype(jnp.float32)
    ones = jnp.ones((h_ref.shape[0], D), jnp.float32)
    dn = (((0,), (0,)), ((), ()))
    seg = lax.dot_general(onehot, y[:, 3 * D:4 * D], dn,
                          preferred_element_type=jnp.float32)
    cnt = lax.dot_general(onehot, ones, dn,
                          preferred_element_type=jnp.float32)

    @pl.when(pl.program_id(0) == 0)
    def _():
        segg_ref[...] = jnp.zeros_like(segg_ref)
        cnta_ref[...] = jnp.zeros_like(cnta_ref)

    segg_ref[...] += seg
    cnta_ref[...] += cnt


def _h_proj(h, ag2d, Wcat, bcat):
    R = 1000
    grid = NA // R
    return pl.pallas_call(
        _k1_body,
        grid=(grid,),
        in_specs=[
            pl.BlockSpec((R, D), lambda i: (i, 0)),
            pl.BlockSpec((R, 1), lambda i: (i, 0)),
            pl.BlockSpec((D, 4 * D), lambda i: (0, 0)),
            pl.BlockSpec((1, 4 * D), lambda i: (0, 0)),
        ],
        out_specs=[
            pl.BlockSpec((R, D), lambda i: (i, 0)),
            pl.BlockSpec((R, D), lambda i: (i, 0)),
            pl.BlockSpec((R, D), lambda i: (i, 0)),
            pl.BlockSpec((NG, D), lambda i: (0, 0)),
            pl.BlockSpec((NG, D), lambda i: (0, 0)),
        ],
        out_shape=[
            jax.ShapeDtypeStruct((NA, D), jnp.float32),
            jax.ShapeDtypeStruct((NA, D), jnp.float32),
            jax.ShapeDtypeStruct((NA, D), jnp.float32),
            jax.ShapeDtypeStruct((NG, D), jnp.float32),
            jax.ShapeDtypeStruct((NG, D), jnp.float32),
        ],
    )(h, ag2d, Wcat, bcat)


def _k2_body(e_ref, bg_ref, w_ref, b_ref, cu_ref,
             becu0_ref, becu1_ref, segh_ref, cntb_ref):
    y = jnp.dot(e_ref[...], w_ref[...],
                preferred_element_type=jnp.float32) + b_ref[...]
    bg = bg_ref[...]
    onehot = (bg == lax.broadcasted_iota(jnp.int32, (1, NG), 1)
              ).astype(jnp.float32)
    becu = y[:, :D] + jnp.dot(onehot, cu_ref[...],
                              preferred_element_type=jnp.float32)
    becu0_ref[...] = becu[:, :H]
    becu1_ref[...] = becu[:, H:]
    he = y[:, D:]
    ones = jnp.ones((e_ref.shape[0], D), jnp.float32)
    dn = (((0,), (0,)), ((), ()))
    seg = lax.dot_general(onehot, he, dn, preferred_element_type=jnp.float32)
    cnt = lax.dot_general(onehot, ones, dn, preferred_element_type=jnp.float32)

    @pl.when(pl.program_id(0) == 0)
    def _():
        segh_ref[...] = jnp.zeros_like(segh_ref)
        cntb_ref[...] = jnp.zeros_like(cntb_ref)

    segh_ref[...] += seg
    cntb_ref[...] += cnt


def _e_proj(e, bg2d, Wcat, bcat, Cu):
    R = 4000
    grid = NB // R
    return pl.pallas_call(
        _k2_body,
        grid=(grid,),
        in_specs=[
            pl.BlockSpec((R, D), lambda i: (i, 0)),
            pl.BlockSpec((R, 1), lambda i: (i, 0)),
            pl.BlockSpec((D, 2 * D), lambda i: (0, 0)),
            pl.BlockSpec((1, 2 * D), lambda i: (0, 0)),
            pl.BlockSpec((NG, D), lambda i: (0, 0)),
        ],
        out_specs=[
            pl.BlockSpec((R, H), lambda i: (i, 0)),
            pl.BlockSpec((R, H), lambda i: (i, 0)),
            pl.BlockSpec((NG, D), lambda i: (0, 0)),
            pl.BlockSpec((NG, D), lambda i: (0, 0)),
        ],
        out_shape=[
            jax.ShapeDtypeStruct((NB, H), jnp.float32),
            jax.ShapeDtypeStruct((NB, H), jnp.float32),
            jax.ShapeDtypeStruct((NG, D), jnp.float32),
            jax.ShapeDtypeStruct((NG, D), jnp.float32),
        ],
    )(e, bg2d, Wcat, bcat, Cu)


def _sc_body(aeh0, aeh1, becu0, becu1, src_hbm, dst_hbm, zeros_hbm,
             ep0, ep1, nd0, nd1,
             idxs0, idxd0, idxs1, idxd1, aes0, aed0, aes1, aed1, bcu0, bcu1,
             ep_v, bufs, bufd, accum, sA0, sB0, sC0, sA1, sB1, sC1):
    cid = lax.axis_index("c")
    sid = lax.axis_index("s")
    wid = sid * NC + cid
    edge_base = wid * EPW
    sets = ((idxs0, idxd0, aes0, aed0, bcu0, sA0, sB0, sC0),
            (idxs1, idxd1, aes1, aed1, bcu1, sA1, sB1, sC1))

    for aeh, becu, ep_out, nd_out in ((aeh0, becu0, ep0, nd0),
                                      (aeh1, becu1, ep1, nd1)):
        @pl.when(sid == 0)
        def _():
            pltpu.sync_copy(zeros_hbm, accum)
        plsc.subcore_barrier()

        def load_idx(st, c):
            base = edge_base + c * K
            pltpu.sync_copy(src_hbm.at[pl.ds(base, K)], st[0])
            pltpu.sync_copy(dst_hbm.at[pl.ds(base, K)], st[1])

        def fire(st, c):
            base = edge_base + c * K
            pltpu.async_copy(aeh.at[st[0]], st[2], st[5])
            pltpu.async_copy(aeh.at[st[1]], st[3], st[6])
            pltpu.async_copy(becu.at[pl.ds(base, K)], st[4], st[7])

        def drain(st):
            pltpu.make_async_copy(aeh.at[st[0]], st[2], st[5]).wait()
            pltpu.make_async_copy(aeh.at[st[1]], st[3], st[6]).wait()
            pltpu.make_async_copy(becu.at[pl.ds(0, K)], st[4], st[7]).wait()

        load_idx(sets[0], 0)
        fire(sets[0], 0)

        def pair(c2, carry):
            for b in range(2):
                cur = sets[b]
                nxt = sets[1 - b]
                c = 2 * c2 + b
                cn = jnp.minimum(c + 1, CHUNKS - 1)
                load_idx(nxt, cn)
                fire(nxt, cn)
                drain(cur)

                def row(i, carry2):
                    def col(j, carry3):
                        s = pl.ds(j * 16, 16)
                        s2 = pl.ds(H + j * 16, 16)
                        a = cur[2][i, s] + cur[3][i, s] + cur[4][i, s]
                        ep_v[i, s] = a
                        sg = 1.0 / (1.0 + jnp.exp(-a))
                        bufs[i, s] = sg * cur[3][i, s2]
                        bufd[i, s] = sg * cur[2][i, s2]
                        bufs[i, s2] = sg
                        bufd[i, s2] = sg
                        return carry3
                    return lax.fori_loop(0, H // 16, col, carry2)
                lax.fori_loop(0, K, row, 0)

                base = edge_base + c * K
                pltpu.sync_copy(ep_v, ep_out.at[pl.ds(base, K)])
                pltpu.sync_copy(bufs, accum.at[cur[0]], add=True)
                pltpu.sync_copy(bufd, accum.at[cur[1]], add=True)
            return carry

        lax.fori_loop(0, CHUNKS // 2, pair, 0)
        drain(sets[0])
        plsc.subcore_barrier()

        @pl.when(sid == 0)
        def _():
            pltpu.sync_copy(accum, nd_out.at[pl.ds(cid * NA, NA)])
        plsc.subcore_barrier()


def _sc_edges(aeh0, aeh1, becu0, becu1, src, dst, zeros):
    fn = pl.kernel(
        _sc_body,
        out_type=[
            jax.ShapeDtypeStruct((NB, H), jnp.float32),
            jax.ShapeDtypeStruct((NB, H), jnp.float32),
            jax.ShapeDtypeStruct((NC * NA, D), jnp.float32),
            jax.ShapeDtypeStruct((NC * NA, D), jnp.float32),
        ],
        mesh=plsc.VectorSubcoreMesh(core_axis_name="c", subcore_axis_name="s"),
        scratch_types=[
            pltpu.VMEM((K,), jnp.int32),
            pltpu.VMEM((K,), jnp.int32),
            pltpu.VMEM((K,), jnp.int32),
            pltpu.VMEM((K,), jnp.int32),
            pltpu.VMEM((K, D), jnp.float32),
            pltpu.VMEM((K, D), jnp.float32),
            pltpu.VMEM((K, D), jnp.float32),
            pltpu.VMEM((K, D), jnp.float32),
            pltpu.VMEM((K, H), jnp.float32),
            pltpu.VMEM((K, H), jnp.float32),
            pltpu.VMEM((K, H), jnp.float32),
            pltpu.VMEM((K, D), jnp.float32),
            pltpu.VMEM((K, D), jnp.float32),
            pltpu.VMEM_SHARED((NA, D), jnp.float32),
            pltpu.SemaphoreType.DMA,
            pltpu.SemaphoreType.DMA,
            pltpu.SemaphoreType.DMA,
            pltpu.SemaphoreType.DMA,
            pltpu.SemaphoreType.DMA,
            pltpu.SemaphoreType.DMA,
        ],
    )
    return fn(aeh0, aeh1, becu0, becu1, src, dst, zeros)


def _k4_body(nd0a_ref, nd0b_ref, nd1a_ref, nd1b_ref, dh_ref, ag_ref,
             fu_ref, sn_ref, hpre_ref, stats_ref):
    num = jnp.concatenate(
        [nd0a_ref[:, :H] + nd0b_ref[:, :H],
         nd1a_ref[:, :H] + nd1b_ref[:, :H]], axis=1)
    den = jnp.concatenate(
        [nd0a_ref[:, H:] + nd0b_ref[:, H:],
         nd1a_ref[:, H:] + nd1b_ref[:, H:]], axis=1)
    gated = num / (den + 1e-6)
    ag = ag_ref[...]
    onehot = (ag == lax.broadcasted_iota(jnp.int32, (1, NG), 1)
              ).astype(jnp.float32)
    fug = jnp.dot(onehot, fu_ref[...], preferred_element_type=jnp.float32)
    x = (dh_ref[...] + gated + fug) * sn_ref[...]
    hpre_ref[...] = x
    s1 = jnp.sum(x, axis=0, keepdims=True)
    s2 = jnp.sum(x * x, axis=0, keepdims=True)
    st = jnp.concatenate([s1, s2, jnp.zeros((6, D), jnp.float32)], axis=0)

    @pl.when(pl.program_id(0) == 0)
    def _():
        stats_ref[...] = jnp.zeros_like(stats_ref)

    stats_ref[...] += st


def _h_assemble(nd0, nd1, Dh, ag2d, Fu, snorm_n):
    R = 1000
    grid = NA // R
    return pl.pallas_call(
        _k4_body,
        grid=(grid,),
        in_specs=[
            pl.BlockSpec((R, D), lambda i: (i, 0)),
            pl.BlockSpec((R, D), lambda i: (i + NA // R, 0)),
            pl.BlockSpec((R, D), lambda i: (i, 0)),
            pl.BlockSpec((R, D), lambda i: (i + NA // R, 0)),
            pl.BlockSpec((R, D), lambda i: (i, 0)),
            pl.BlockSpec((R, 1), lambda i: (i, 0)),
            pl.BlockSpec((NG, D), lambda i: (0, 0)),
            pl.BlockSpec((R, 1), lambda i: (i, 0)),
        ],
        out_specs=[
            pl.BlockSpec((R, D), lambda i: (i, 0)),
            pl.BlockSpec((8, D), lambda i: (0, 0)),
        ],
        out_shape=[
            jax.ShapeDtypeStruct((NA, D), jnp.float32),
            jax.ShapeDtypeStruct((8, D), jnp.float32),
        ],
    )(nd0, nd0, nd1, nd1, Dh, ag2d, Fu, snorm_n)


def _k5_body(ep0_ref, ep1_ref, sn_ref, stats_ref):
    x = jnp.concatenate([ep0_ref[...], ep1_ref[...]], axis=1) * sn_ref[...]
    s1 = jnp.sum(x, axis=0, keepdims=True)
    s2 = jnp.sum(x * x, axis=0, keepdims=True)
    st = jnp.concatenate([s1, s2, jnp.zeros((6, D), jnp.float32)], axis=0)

    @pl.when(pl.program_id(0) == 0)
    def _():
        stats_ref[...] = jnp.zeros_like(stats_ref)

    stats_ref[...] += st


def _e_stats(ep0, ep1, snorm_e):
    R = 4000
    grid = NB // R
    return pl.pallas_call(
        _k5_body,
        grid=(grid,),
        in_specs=[
            pl.BlockSpec((R, H), lambda i: (i, 0)),
            pl.BlockSpec((R, H), lambda i: (i, 0)),
            pl.BlockSpec((R, 1), lambda i: (i, 0)),
        ],
        out_specs=pl.BlockSpec((8, D), lambda i: (0, 0)),
        out_shape=jax.ShapeDtypeStruct((8, D), jnp.float32),
    )(ep0, ep1, snorm_e)


def _bn_elu(x, stats, n, gamma, beta):
    m = stats[0:1, :] / n
    v = stats[1:2, :] / n - m * m
    y = gamma * (x - m) * lax.rsqrt(v + 1e-5) + beta
    return jnp.where(y > 0, y, jnp.exp(jnp.minimum(y, 0.0)) - 1.0)


def _k6_body(ep0_ref, ep1_ref, sn_ref, stats_ref, g_ref, b_ref, out_ref):
    x = jnp.concatenate([ep0_ref[...], ep1_ref[...]], axis=1) * sn_ref[...]
    out_ref[...] = _bn_elu(x, stats_ref[...], float(NB), g_ref[...], b_ref[...])


def _e_apply(ep0, ep1, snorm_e, stats, gamma, beta):
    R = 4000
    grid = NB // R
    return pl.pallas_call(
        _k6_body,
        grid=(grid,),
        in_specs=[
            pl.BlockSpec((R, H), lambda i: (i, 0)),
            pl.BlockSpec((R, H), lambda i: (i, 0)),
            pl.BlockSpec((R, 1), lambda i: (i, 0)),
            pl.BlockSpec((8, D), lambda i: (0, 0)),
            pl.BlockSpec((1, D), lambda i: (0, 0)),
            pl.BlockSpec((1, D), lambda i: (0, 0)),
        ],
        out_specs=pl.BlockSpec((R, D), lambda i: (i, 0)),
        out_shape=jax.ShapeDtypeStruct((NB, D), jnp.float32),
    )(ep0, ep1, snorm_e, stats, gamma, beta)


def _k7a_body(x_ref, stats_ref, g_ref, b_ref, out_ref):
    out_ref[...] = _bn_elu(x_ref[...], stats_ref[...], float(NA),
                           g_ref[...], b_ref[...])


def _h_apply(hpre, stats, gamma, beta):
    R = 1000
    grid = NA // R
    return pl.pallas_call(
        _k7a_body,
        grid=(grid,),
        in_specs=[
            pl.BlockSpec((R, D), lambda i: (i, 0)),
            pl.BlockSpec((8, D), lambda i: (0, 0)),
            pl.BlockSpec((1, D), lambda i: (0, 0)),
            pl.BlockSpec((1, D), lambda i: (0, 0)),
        ],
        out_specs=pl.BlockSpec((R, D), lambda i: (i, 0)),
        out_shape=jax.ShapeDtypeStruct((NA, D), jnp.float32),
    )(hpre, stats, gamma, beta)


def _k7b_body(u_ref, wi_ref, bi_ref, segg_ref, cnta_ref, segh_ref, cntb_ref,
              g_ref, b_ref, out_ref):
    iu = jnp.dot(u_ref[...], wi_ref[...],
                 preferred_element_type=jnp.float32) + bi_ref[...]
    x = (segg_ref[...] / jnp.maximum(cnta_ref[...], 1.0)
         + segh_ref[...] / jnp.maximum(cntb_ref[...], 1.0) + iu)
    m = jnp.mean(x, axis=0, keepdims=True)
    v = jnp.mean(x * x, axis=0, keepdims=True) - m * m
    y = g_ref[...] * (x - m) * lax.rsqrt(v + 1e-5) + b_ref[...]
    out_ref[...] = jnp.where(y > 0, y, jnp.exp(jnp.minimum(y, 0.0)) - 1.0)


def _u_update(u, WI, bI, segG, cntA, segH, cntB, gamma, beta):
    return pl.pallas_call(
        _k7b_body,
        out_shape=jax.ShapeDtypeStruct((NG, D), jnp.float32),
    )(u, WI, bI, segG, cntA, segH, cntB, gamma, beta)


@jax.jit
def kernel(h, e, u, bond_src, bond_dst, atom_graph, bond_graph, snorm_n,
           snorm_e, WA, bA, WB, bB, WC, bC, WD, bD, WE, bE, WF, bF, WG, bG,
           WH, bH, WI, bI, gamma_h, beta_h, gamma_e, beta_e, gamma_u, beta_u):
    ag2d = atom_graph.reshape(NA, 1)
    bg2d = bond_graph.reshape(NB, 1)

    cufu = _u_proj(u, jnp.concatenate([WC, WF], axis=1),
                   jnp.concatenate([bC, bF]).reshape(1, 2 * D))
    Cu = cufu[:, :D]
    Fu = cufu[:, D:]

    Wh = jnp.concatenate([WA, WD, WE, WG], axis=1)
    bh = jnp.concatenate([bA, bD, bE, bG]).reshape(1, 4 * D)
    Dh, aeh0, aeh1, segG, cntA = _h_proj(h, ag2d, Wh, bh)

    We = jnp.concatenate([WB, WH], axis=1)
    be = jnp.concatenate([bB, bH]).reshape(1, 2 * D)
    becu0, becu1, segH, cntB = _e_proj(e, bg2d, We, be, Cu)

    zeros = jnp.zeros((NA, D), jnp.float32)
    ep0, ep1, nd0, nd1 = _sc_edges(aeh0, aeh1, becu0, becu1,
                                   bond_src, bond_dst, zeros)

    hpre, hstats = _h_assemble(nd0, nd1, Dh, ag2d, Fu, snorm_n)

    estats = _e_stats(ep0, ep1, snorm_e)
    e_new = _e_apply(ep0, ep1, snorm_e, estats,
                     gamma_e.reshape(1, D), beta_e.reshape(1, D))

    h_new = _h_apply(hpre, hstats, gamma_h.reshape(1, D), beta_h.reshape(1, D))

    u_new = _u_update(u, WI, bI.reshape(1, D), segG, cntA, segH, cntB,
                      gamma_u.reshape(1, D), beta_u.reshape(1, D))

    return h_new, e_new, u_new


# speedup vs baseline: 5.0995x; 1.0473x over previous
import functools

import jax
import jax.numpy as jnp
from jax import lax
from jax.experimental import pallas as pl
from jax.experimental.pallas import tpu as pltpu
from jax.experimental.pallas import tpu_sc as plsc

NA = 10000
NB = 320000
NG = 64
D = 128
H = 64

NC = 2
NS = 16
NW = NC * NS
EPW = NB // NW
K = 40
CHUNKS = EPW // K
RPS = NA // NS


def _k0_body(u_ref, w_ref, b_ref, out_ref):
    out_ref[...] = jnp.dot(u_ref[...], w_ref[...],
                           preferred_element_type=jnp.float32) + b_ref[...]


def _u_proj(u, Wcat, bcat):
    return pl.pallas_call(
        _k0_body,
        out_shape=jax.ShapeDtypeStruct((NG, Wcat.shape[1]), jnp.float32),
    )(u, Wcat, bcat)


def _k1_body(h_ref, ag_ref, w_ref, b_ref, dh_ref, aeh0_ref, aeh1_ref,
             segg_ref, cnta_ref):
    y = jnp.dot(h_ref[...], w_ref[...],
                preferred_element_type=jnp.float32) + b_ref[...]
    dh_ref[...] = y[:, D:2 * D]
    aeh0_ref[...] = jnp.concatenate(
        [y[:, 0:H], y[:, 2 * D:2 * D + H]], axis=1)
    aeh1_ref[...] = jnp.concatenate(
        [y[:, H:D], y[:, 2 * D + H:3 * D]], axis=1)
    ag = ag_ref[...]
    onehot = (ag == lax.broadcasted_iota(jnp.int32, (1, NG), 1)
              ).astype(jnp.float32)
    ones = jnp.ones((h_ref.shape[0], D), jnp.float32)
    dn = (((0,), (0,)), ((), ()))
    seg = lax.dot_general(onehot, y[:, 3 * D:4 * D], dn,
                          preferred_element_type=jnp.float32)
    cnt = lax.dot_general(onehot, ones, dn,
                          preferred_element_type=jnp.float32)

    @pl.when(pl.program_id(0) == 0)
    def _():
        segg_ref[...] = jnp.zeros_like(segg_ref)
        cnta_ref[...] = jnp.zeros_like(cnta_ref)

    segg_ref[...] += seg
    cnta_ref[...] += cnt


def _h_proj(h, ag2d, Wcat, bcat):
    R = 1000
    grid = NA // R
    return pl.pallas_call(
        _k1_body,
        grid=(grid,),
        in_specs=[
            pl.BlockSpec((R, D), lambda i: (i, 0)),
            pl.BlockSpec((R, 1), lambda i: (i, 0)),
            pl.BlockSpec((D, 4 * D), lambda i: (0, 0)),
            pl.BlockSpec((1, 4 * D), lambda i: (0, 0)),
        ],
        out_specs=[
            pl.BlockSpec((R, D), lambda i: (i, 0)),
            pl.BlockSpec((R, D), lambda i: (i, 0)),
            pl.BlockSpec((R, D), lambda i: (i, 0)),
            pl.BlockSpec((NG, D), lambda i: (0, 0)),
            pl.BlockSpec((NG, D), lambda i: (0, 0)),
        ],
        out_shape=[
            jax.ShapeDtypeStruct((NA, D), jnp.float32),
            jax.ShapeDtypeStruct((NA, D), jnp.float32),
            jax.ShapeDtypeStruct((NA, D), jnp.float32),
            jax.ShapeDtypeStruct((NG, D), jnp.float32),
            jax.ShapeDtypeStruct((NG, D), jnp.float32),
        ],
    )(h, ag2d, Wcat, bcat)


def _k2_body(e_ref, bg_ref, w_ref, b_ref, cu_ref,
             becu0_ref, becu1_ref, segh_ref, cntb_ref):
    y = jnp.dot(e_ref[...], w_ref[...],
                preferred_element_type=jnp.float32) + b_ref[...]
    bg = bg_ref[...]
    onehot = (bg == lax.broadcasted_iota(jnp.int32, (1, NG), 1)
              ).astype(jnp.float32)
    becu = y[:, :D] + jnp.dot(onehot, cu_ref[...],
                              preferred_element_type=jnp.float32)
    becu0_ref[...] = becu[:, :H]
    becu1_ref[...] = becu[:, H:]
    he = y[:, D:]
    ones = jnp.ones((e_ref.shape[0], D), jnp.float32)
    dn = (((0,), (0,)), ((), ()))
    seg = lax.dot_general(onehot, he, dn, preferred_element_type=jnp.float32)
    cnt = lax.dot_general(onehot, ones, dn, preferred_element_type=jnp.float32)

    @pl.when(pl.program_id(0) == 0)
    def _():
        segh_ref[...] = jnp.zeros_like(segh_ref)
        cntb_ref[...] = jnp.zeros_like(cntb_ref)

    segh_ref[...] += seg
    cntb_ref[...] += cnt


def _e_proj(e, bg2d, Wcat, bcat, Cu):
    R = 4000
    grid = NB // R
    return pl.pallas_call(
        _k2_body,
        grid=(grid,),
        in_specs=[
            pl.BlockSpec((R, D), lambda i: (i, 0)),
            pl.BlockSpec((R, 1), lambda i: (i, 0)),
            pl.BlockSpec((D, 2 * D), lambda i: (0, 0)),
            pl.BlockSpec((1, 2 * D), lambda i: (0, 0)),
            pl.BlockSpec((NG, D), lambda i: (0, 0)),
        ],
        out_specs=[
            pl.BlockSpec((R, H), lambda i: (i, 0)),
            pl.BlockSpec((R, H), lambda i: (i, 0)),
            pl.BlockSpec((NG, D), lambda i: (0, 0)),
            pl.BlockSpec((NG, D), lambda i: (0, 0)),
        ],
        out_shape=[
            jax.ShapeDtypeStruct((NB, H), jnp.float32),
            jax.ShapeDtypeStruct((NB, H), jnp.float32),
            jax.ShapeDtypeStruct((NG, D), jnp.float32),
            jax.ShapeDtypeStruct((NG, D), jnp.float32),
        ],
    )(e, bg2d, Wcat, bcat, Cu)


def _sc_body(aeh0, aeh1, becu0, becu1, src_hbm, dst_hbm, zeros_hbm,
             ep0, ep1, nd0, nd1,
             idxs0, idxd0, idxs1, idxd1, aes0, aed0, aes1, aed1, bcu0, bcu1,
             ep_v, bufs, bufd, accum, sA0, sB0, sC0, sA1, sB1, sC1, sW):
    cid = lax.axis_index("c")
    sid = lax.axis_index("s")
    wid = sid * NC + cid
    edge_base = wid * EPW
    sets = ((idxs0, idxd0, aes0, aed0, bcu0, sA0, sB0, sC0),
            (idxs1, idxd1, aes1, aed1, bcu1, sA1, sB1, sC1))

    for aeh, becu, ep_out, nd_out in ((aeh0, becu0, ep0, nd0),
                                      (aeh1, becu1, ep1, nd1)):
        @pl.when(sid == 0)
        def _():
            pltpu.sync_copy(zeros_hbm, accum)
        plsc.subcore_barrier()

        def load_idx(st, c):
            base = edge_base + c * K
            pltpu.sync_copy(src_hbm.at[pl.ds(base, K)], st[0])
            pltpu.sync_copy(dst_hbm.at[pl.ds(base, K)], st[1])

        def fire(st, c):
            base = edge_base + c * K
            pltpu.async_copy(aeh.at[st[0]], st[2], st[5])
            pltpu.async_copy(aeh.at[st[1]], st[3], st[6])
            pltpu.async_copy(becu.at[pl.ds(base, K)], st[4], st[7])

        def drain(st):
            pltpu.make_async_copy(aeh.at[st[0]], st[2], st[5]).wait()
            pltpu.make_async_copy(aeh.at[st[1]], st[3], st[6]).wait()
            pltpu.make_async_copy(becu.at[pl.ds(0, K)], st[4], st[7]).wait()

        load_idx(sets[0], 0)
        fire(sets[0], 0)

        def pair(c2, carry):
            for b in range(2):
                cur = sets[b]
                nxt = sets[1 - b]
                c = 2 * c2 + b
                cn = jnp.minimum(c + 1, CHUNKS - 1)
                load_idx(nxt, cn)
                fire(nxt, cn)
                drain(cur)

                def drain_writes():
                    pltpu.make_async_copy(
                        ep_v, ep_out.at[pl.ds(0, K)], sW).wait()

                if b == 1:
                    drain_writes()
                else:
                    pl.when(c2 > 0)(drain_writes)

                def row(i, carry2):
                    def col(j, carry3):
                        s = pl.ds(j * 16, 16)
                        s2 = pl.ds(H + j * 16, 16)
                        a = cur[2][i, s] + cur[3][i, s] + cur[4][i, s]
                        ep_v[i, s] = a
                        sg = 1.0 / (1.0 + jnp.exp(-a))
                        bufs[i, s] = sg * cur[3][i, s2]
                        bufd[i, s] = sg * cur[2][i, s2]
                        bufs[i, s2] = sg
                        bufd[i, s2] = sg
                        return carry3
                    return lax.fori_loop(0, H // 16, col, carry2)
                lax.fori_loop(0, K, row, 0)

                base = edge_base + c * K
                pltpu.async_copy(ep_v, ep_out.at[pl.ds(base, K)], sW)
                pltpu.sync_copy(bufs, accum.at[cur[0]], add=True)
                pltpu.sync_copy(bufd, accum.at[cur[1]], add=True)
            return carry

        lax.fori_loop(0, CHUNKS // 2, pair, 0)
        drain(sets[0])
        pltpu.make_async_copy(ep_v, ep_out.at[pl.ds(0, K)], sW).wait()
        plsc.subcore_barrier()

        @pl.when(sid == 0)
        def _():
            pltpu.sync_copy(accum, nd_out.at[pl.ds(cid * NA, NA)])
        plsc.subcore_barrier()


def _sc_edges(aeh0, aeh1, becu0, becu1, src, dst, zeros):
    fn = pl.kernel(
        _sc_body,
        out_type=[
            jax.ShapeDtypeStruct((NB, H), jnp.float32),
            jax.ShapeDtypeStruct((NB, H), jnp.float32),
            jax.ShapeDtypeStruct((NC * NA, D), jnp.float32),
            jax.ShapeDtypeStruct((NC * NA, D), jnp.float32),
        ],
        mesh=plsc.VectorSubcoreMesh(core_axis_name="c", subcore_axis_name="s"),
        scratch_types=[
            pltpu.VMEM((K,), jnp.int32),
            pltpu.VMEM((K,), jnp.int32),
            pltpu.VMEM((K,), jnp.int32),
            pltpu.VMEM((K,), jnp.int32),
            pltpu.VMEM((K, D), jnp.float32),
            pltpu.VMEM((K, D), jnp.float32),
            pltpu.VMEM((K, D), jnp.float32),
            pltpu.VMEM((K, D), jnp.float32),
            pltpu.VMEM((K, H), jnp.float32),
            pltpu.VMEM((K, H), jnp.float32),
            pltpu.VMEM((K, H), jnp.float32),
            pltpu.VMEM((K, D), jnp.float32),
            pltpu.VMEM((K, D), jnp.float32),
            pltpu.VMEM_SHARED((NA, D), jnp.float32),
            pltpu.SemaphoreType.DMA,
            pltpu.SemaphoreType.DMA,
            pltpu.SemaphoreType.DMA,
            pltpu.SemaphoreType.DMA,
            pltpu.SemaphoreType.DMA,
            pltpu.SemaphoreType.DMA,
            pltpu.SemaphoreType.DMA,
        ],
    )
    return fn(aeh0, aeh1, becu0, becu1, src, dst, zeros)


def _k4_body(nd0a_ref, nd0b_ref, nd1a_ref, nd1b_ref, dh_ref, ag_ref,
             fu_ref, sn_ref, hpre_ref, stats_ref):
    num = jnp.concatenate(
        [nd0a_ref[:, :H] + nd0b_ref[:, :H],
         nd1a_ref[:, :H] + nd1b_ref[:, :H]], axis=1)
    den = jnp.concatenate(
        [nd0a_ref[:, H:] + nd0b_ref[:, H:],
         nd1a_ref[:, H:] + nd1b_ref[:, H:]], axis=1)
    gated = num / (den + 1e-6)
    ag = ag_ref[...]
    onehot = (ag == lax.broadcasted_iota(jnp.int32, (1, NG), 1)
              ).astype(jnp.float32)
    fug = jnp.dot(onehot, fu_ref[...], preferred_element_type=jnp.float32)
    x = (dh_ref[...] + gated + fug) * sn_ref[...]
    hpre_ref[...] = x
    s1 = jnp.sum(x, axis=0, keepdims=True)
    s2 = jnp.sum(x * x, axis=0, keepdims=True)
    st = jnp.concatenate([s1, s2, jnp.zeros((6, D), jnp.float32)], axis=0)

    @pl.when(pl.program_id(0) == 0)
    def _():
        stats_ref[...] = jnp.zeros_like(stats_ref)

    stats_ref[...] += st


def _h_assemble(nd0, nd1, Dh, ag2d, Fu, snorm_n):
    R = 1000
    grid = NA // R
    return pl.pallas_call(
        _k4_body,
        grid=(grid,),
        in_specs=[
            pl.BlockSpec((R, D), lambda i: (i, 0)),
            pl.BlockSpec((R, D), lambda i: (i + NA // R, 0)),
            pl.BlockSpec((R, D), lambda i: (i, 0)),
            pl.BlockSpec((R, D), lambda i: (i + NA // R, 0)),
            pl.BlockSpec((R, D), lambda i: (i, 0)),
            pl.BlockSpec((R, 1), lambda i: (i, 0)),
            pl.BlockSpec((NG, D), lambda i: (0, 0)),
            pl.BlockSpec((R, 1), lambda i: (i, 0)),
        ],
        out_specs=[
            pl.BlockSpec((R, D), lambda i: (i, 0)),
            pl.BlockSpec((8, D), lambda i: (0, 0)),
        ],
        out_shape=[
            jax.ShapeDtypeStruct((NA, D), jnp.float32),
            jax.ShapeDtypeStruct((8, D), jnp.float32),
        ],
    )(nd0, nd0, nd1, nd1, Dh, ag2d, Fu, snorm_n)


def _k5_body(ep0_ref, ep1_ref, sn_ref, stats_ref):
    x = jnp.concatenate([ep0_ref[...], ep1_ref[...]], axis=1) * sn_ref[...]
    s1 = jnp.sum(x, axis=0, keepdims=True)
    s2 = jnp.sum(x * x, axis=0, keepdims=True)
    st = jnp.concatenate([s1, s2, jnp.zeros((6, D), jnp.float32)], axis=0)

    @pl.when(pl.program_id(0) == 0)
    def _():
        stats_ref[...] = jnp.zeros_like(stats_ref)

    stats_ref[...] += st


def _e_stats(ep0, ep1, snorm_e):
    R = 4000
    grid = NB // R
    return pl.pallas_call(
        _k5_body,
        grid=(grid,),
        in_specs=[
            pl.BlockSpec((R, H), lambda i: (i, 0)),
            pl.BlockSpec((R, H), lambda i: (i, 0)),
            pl.BlockSpec((R, 1), lambda i: (i, 0)),
        ],
        out_specs=pl.BlockSpec((8, D), lambda i: (0, 0)),
        out_shape=jax.ShapeDtypeStruct((8, D), jnp.float32),
    )(ep0, ep1, snorm_e)


def _bn_elu(x, stats, n, gamma, beta):
    m = stats[0:1, :] / n
    v = stats[1:2, :] / n - m * m
    y = gamma * (x - m) * lax.rsqrt(v + 1e-5) + beta
    return jnp.where(y > 0, y, jnp.exp(jnp.minimum(y, 0.0)) - 1.0)


def _k6_body(ep0_ref, ep1_ref, sn_ref, stats_ref, g_ref, b_ref, out_ref):
    x = jnp.concatenate([ep0_ref[...], ep1_ref[...]], axis=1) * sn_ref[...]
    out_ref[...] = _bn_elu(x, stats_ref[...], float(NB), g_ref[...], b_ref[...])


def _e_apply(ep0, ep1, snorm_e, stats, gamma, beta):
    R = 4000
    grid = NB // R
    return pl.pallas_call(
        _k6_body,
        grid=(grid,),
        in_specs=[
            pl.BlockSpec((R, H), lambda i: (i, 0)),
            pl.BlockSpec((R, H), lambda i: (i, 0)),
            pl.BlockSpec((R, 1), lambda i: (i, 0)),
            pl.BlockSpec((8, D), lambda i: (0, 0)),
            pl.BlockSpec((1, D), lambda i: (0, 0)),
            pl.BlockSpec((1, D), lambda i: (0, 0)),
        ],
        out_specs=pl.BlockSpec((R, D), lambda i: (i, 0)),
        out_shape=jax.ShapeDtypeStruct((NB, D), jnp.float32),
    )(ep0, ep1, snorm_e, stats, gamma, beta)


def _k7a_body(x_ref, stats_ref, g_ref, b_ref, out_ref):
    out_ref[...] = _bn_elu(x_ref[...], stats_ref[...], float(NA),
                           g_ref[...], b_ref[...])


def _h_apply(hpre, stats, gamma, beta):
    R = 1000
    grid = NA // R
    return pl.pallas_call(
        _k7a_body,
        grid=(grid,),
        in_specs=[
            pl.BlockSpec((R, D), lambda i: (i, 0)),
            pl.BlockSpec((8, D), lambda i: (0, 0)),
            pl.BlockSpec((1, D), lambda i: (0, 0)),
            pl.BlockSpec((1, D), lambda i: (0, 0)),
        ],
        out_specs=pl.BlockSpec((R, D), lambda i: (i, 0)),
        out_shape=jax.ShapeDtypeStruct((NA, D), jnp.float32),
    )(hpre, stats, gamma, beta)


def _k7b_body(u_ref, wi_ref, bi_ref, segg_ref, cnta_ref, segh_ref, cntb_ref,
              g_ref, b_ref, out_ref):
    iu = jnp.dot(u_ref[...], wi_ref[...],
                 preferred_element_type=jnp.float32) + bi_ref[...]
    x = (segg_ref[...] / jnp.maximum(cnta_ref[...], 1.0)
         + segh_ref[...] / jnp.maximum(cntb_ref[...], 1.0) + iu)
    m = jnp.mean(x, axis=0, keepdims=True)
    v = jnp.mean(x * x, axis=0, keepdims=True) - m * m
    y = g_ref[...] * (x - m) * lax.rsqrt(v + 1e-5) + b_ref[...]
    out_ref[...] = jnp.where(y > 0, y, jnp.exp(jnp.minimum(y, 0.0)) - 1.0)


def _u_update(u, WI, bI, segG, cntA, segH, cntB, gamma, beta):
    return pl.pallas_call(
        _k7b_body,
        out_shape=jax.ShapeDtypeStruct((NG, D), jnp.float32),
    )(u, WI, bI, segG, cntA, segH, cntB, gamma, beta)


@jax.jit
def kernel(h, e, u, bond_src, bond_dst, atom_graph, bond_graph, snorm_n,
           snorm_e, WA, bA, WB, bB, WC, bC, WD, bD, WE, bE, WF, bF, WG, bG,
           WH, bH, WI, bI, gamma_h, beta_h, gamma_e, beta_e, gamma_u, beta_u):
    ag2d = atom_graph.reshape(NA, 1)
    bg2d = bond_graph.reshape(NB, 1)

    cufu = _u_proj(u, jnp.concatenate([WC, WF], axis=1),
                   jnp.concatenate([bC, bF]).reshape(1, 2 * D))
    Cu = cufu[:, :D]
    Fu = cufu[:, D:]

    Wh = jnp.concatenate([WA, WD, WE, WG], axis=1)
    bh = jnp.concatenate([bA, bD, bE, bG]).reshape(1, 4 * D)
    Dh, aeh0, aeh1, segG, cntA = _h_proj(h, ag2d, Wh, bh)

    We = jnp.concatenate([WB, WH], axis=1)
    be = jnp.concatenate([bB, bH]).reshape(1, 2 * D)
    becu0, becu1, segH, cntB = _e_proj(e, bg2d, We, be, Cu)

    zeros = jnp.zeros((NA, D), jnp.float32)
    ep0, ep1, nd0, nd1 = _sc_edges(aeh0, aeh1, becu0, becu1,
                                   bond_src, bond_dst, zeros)

    hpre, hstats = _h_assemble(nd0, nd1, Dh, ag2d, Fu, snorm_n)

    estats = _e_stats(ep0, ep1, snorm_e)
    e_new = _e_apply(ep0, ep1, snorm_e, estats,
                     gamma_e.reshape(1, D), beta_e.reshape(1, D))

    h_new = _h_apply(hpre, hstats, gamma_h.reshape(1, D), beta_h.reshape(1, D))

    u_new = _u_update(u, WI, bI.reshape(1, D), segG, cntA, segH, cntB,
                      gamma_u.reshape(1, D), beta_u.reshape(1, D))

    return h_new, e_new, u_new
